# baseline (speedup 1.0000x reference)
import functools

import jax
import jax.numpy as jnp
from jax import lax
from jax.experimental import pallas as pl
from jax.experimental.pallas import tpu as pltpu
from jax.experimental.pallas import tpu_sc as plsc

_SEP_TOKEN_IDX = 5
_SEQ_BLK = 512
_LANES = 16
_NW = 32
_CHUNK = 1024
_CHUNKS_PER_COL = 8


def _sc_seg_ids(tok_ref, out_ref, tokv, segv):
    c = lax.axis_index("c")
    s = lax.axis_index("s")
    w = c * 16 + s
    nvec = _CHUNK // _LANES
    col_start = (w // _CHUNKS_PER_COL) * _CHUNKS_PER_COL

    def _count_chunk(chunk_idx):
        pltpu.sync_copy(tok_ref.at[pl.ds(chunk_idx * _CHUNK, _CHUNK)], tokv)

        def _body(i, acc):
            tv = tokv[pl.ds(i * _LANES, _LANES)]
            return acc + plsc.all_reduce_population_count(tv == _SEP_TOKEN_IDX)

        return lax.fori_loop(0, nvec, _body, jnp.zeros((_LANES,), jnp.int32))

    base = jnp.zeros((_LANES,), jnp.int32)
    for j in range(_CHUNKS_PER_COL - 1):
        chunk_idx = col_start + j
        valid = (chunk_idx < w).astype(jnp.int32)
        base = base + _count_chunk(chunk_idx) * valid

    pltpu.sync_copy(tok_ref.at[pl.ds(w * _CHUNK, _CHUNK)], tokv)

    def _body(i, carry):
        tv = tokv[pl.ds(i * _LANES, _LANES)]
        m = tv == _SEP_TOKEN_IDX
        si = m.astype(jnp.int32)
        seg = plsc.cumsum(si) - si + carry
        segv[pl.ds(i * _LANES, _LANES)] = jnp.minimum(seg, 2).astype(jnp.float32)
        return carry + plsc.all_reduce_population_count(m)

    lax.fori_loop(0, nvec, _body, base)
    pltpu.sync_copy(segv, out_ref.at[pl.ds(w * _CHUNK, _CHUNK)])


def _tc_add_kernel(seg_ref, x_ref, emb_ref, out_ref):
    seg = jnp.transpose(seg_ref[...])
    nseg, hid = emb_ref.shape
    seg3 = seg[:, :, None]
    e0 = emb_ref[0, :].reshape(1, 1, hid)
    e1 = emb_ref[1, :].reshape(1, 1, hid)
    e2 = emb_ref[2, :].reshape(1, 1, hid)
    emb = jnp.where(seg3 == 0, e0, jnp.where(seg3 == 1, e1, e2))
    out_ref[...] = x_ref[...] + emb


def kernel(x, tokens, emb_table):
    seq, batch, hid = x.shape
    nseg = emb_table.shape[0]
    n = seq * batch
    tok_t = jnp.transpose(tokens.astype(jnp.int32)).reshape(n)

    sc = pl.kernel(
        _sc_seg_ids,
        mesh=plsc.VectorSubcoreMesh(core_axis_name="c", subcore_axis_name="s"),
        compiler_params=pltpu.CompilerParams(needs_layout_passes=False),
        out_type=jax.ShapeDtypeStruct((n,), jnp.float32),
        scratch_types=[
            pltpu.VMEM((_CHUNK,), jnp.int32),
            pltpu.VMEM((_CHUNK,), jnp.float32),
        ],
    )
    seg_t = sc(tok_t).reshape(batch, seq)

    grid = seq // _SEQ_BLK
    return pl.pallas_call(
        _tc_add_kernel,
        grid=(grid,),
        in_specs=[
            pl.BlockSpec((batch, _SEQ_BLK), lambda i: (0, i)),
            pl.BlockSpec((_SEQ_BLK, batch, hid), lambda i: (i, 0, 0)),
            pl.BlockSpec((nseg, hid), lambda i: (0, 0)),
        ],
        out_specs=pl.BlockSpec((_SEQ_BLK, batch, hid), lambda i: (i, 0, 0)),
        out_shape=jax.ShapeDtypeStruct(x.shape, x.dtype),
    )(seg_t, x, emb_table)


# speedup vs baseline: 1.0018x; 1.0018x over previous
import functools

import jax
import jax.numpy as jnp
from jax import lax
from jax.experimental import pallas as pl
from jax.experimental.pallas import tpu as pltpu
from jax.experimental.pallas import tpu_sc as plsc

_SEP_TOKEN_IDX = 5
_SEQ_BLK = 512
_LANES = 16
_NW = 32
_CHUNK = 1024
_CHUNKS_PER_COL = 8


def _sc_seg_ids(tok_ref, out_ref, tokv, segv):
    c = lax.axis_index("c")
    s = lax.axis_index("s")
    w = c * 16 + s
    nvec = _CHUNK // _LANES
    col_start = (w // _CHUNKS_PER_COL) * _CHUNKS_PER_COL

    def _count_chunk(chunk_idx):
        pltpu.sync_copy(tok_ref.at[pl.ds(chunk_idx * _CHUNK, _CHUNK)], tokv)

        def _body(i, acc):
            tv = tokv[pl.ds(i * _LANES, _LANES)]
            return acc + plsc.all_reduce_population_count(tv == _SEP_TOKEN_IDX)

        return lax.fori_loop(0, nvec, _body, jnp.zeros((_LANES,), jnp.int32))

    base = jnp.zeros((_LANES,), jnp.int32)
    for j in range(_CHUNKS_PER_COL - 1):
        chunk_idx = col_start + j
        valid = (chunk_idx < w).astype(jnp.int32)
        base = base + _count_chunk(chunk_idx) * valid

    pltpu.sync_copy(tok_ref.at[pl.ds(w * _CHUNK, _CHUNK)], tokv)

    def _body(i, carry):
        tv = tokv[pl.ds(i * _LANES, _LANES)]
        m = tv == _SEP_TOKEN_IDX
        si = m.astype(jnp.int32)
        seg = plsc.cumsum(si) - si + carry
        segv[pl.ds(i * _LANES, _LANES)] = jnp.minimum(seg, 2).astype(jnp.float32)
        return carry + plsc.all_reduce_population_count(m)

    lax.fori_loop(0, nvec, _body, base)
    pltpu.sync_copy(segv, out_ref.at[pl.ds(w * _CHUNK, _CHUNK)])


def _tc_add_kernel(seg_ref, x_ref, emb_ref, out_ref):
    seg = jnp.transpose(seg_ref[...])
    nseg, hid = emb_ref.shape
    seg3 = seg[:, :, None]
    e0 = emb_ref[0, :].reshape(1, 1, hid)
    e1 = emb_ref[1, :].reshape(1, 1, hid)
    e2 = emb_ref[2, :].reshape(1, 1, hid)
    emb = jnp.where(seg3 == 0, e0, jnp.where(seg3 == 1, e1, e2))
    out_ref[...] = x_ref[...] + emb


def kernel(x, tokens, emb_table):
    seq, batch, hid = x.shape
    nseg = emb_table.shape[0]
    n = seq * batch
    tok_t = jnp.transpose(tokens.astype(jnp.int32)).reshape(n)

    sc = pl.kernel(
        _sc_seg_ids,
        mesh=plsc.VectorSubcoreMesh(core_axis_name="c", subcore_axis_name="s"),
        compiler_params=pltpu.CompilerParams(needs_layout_passes=False),
        out_type=jax.ShapeDtypeStruct((n,), jnp.float32),
        scratch_types=[
            pltpu.VMEM((_CHUNK,), jnp.int32),
            pltpu.VMEM((_CHUNK,), jnp.float32),
        ],
    )
    seg_t = sc(tok_t).reshape(batch, seq)

    grid = seq // _SEQ_BLK
    return pl.pallas_call(
        _tc_add_kernel,
        grid=(grid,),
        compiler_params=pltpu.CompilerParams(
            dimension_semantics=("parallel",)
        ),
        in_specs=[
            pl.BlockSpec((batch, _SEQ_BLK), lambda i: (0, i)),
            pl.BlockSpec((_SEQ_BLK, batch, hid), lambda i: (i, 0, 0)),
            pl.BlockSpec((nseg, hid), lambda i: (0, 0)),
        ],
        out_specs=pl.BlockSpec((_SEQ_BLK, batch, hid), lambda i: (i, 0, 0)),
        out_shape=jax.ShapeDtypeStruct(x.shape, x.dtype),
    )(seg_t, x, emb_table)


# speedup vs baseline: 1.0370x; 1.0351x over previous
import functools

import jax
import jax.numpy as jnp
from jax import lax
from jax.experimental import pallas as pl
from jax.experimental.pallas import tpu as pltpu
from jax.experimental.pallas import tpu_sc as plsc

_SEP_TOKEN_IDX = 5
_SEQ_BLK = 512
_LANES = 16
_NW = 32
_CHUNK = 1024
_CHUNKS_PER_COL = 8


def _sc_seg_ids(tok_ref, out_ref, tokv, segv):
    c = lax.axis_index("c")
    s = lax.axis_index("s")
    w = c * 16 + s
    nvec = _CHUNK // _LANES
    col = w // _CHUNKS_PER_COL
    r = w % _CHUNKS_PER_COL
    col_len = _CHUNK * _CHUNKS_PER_COL

    pltpu.sync_copy(tok_ref.at[pl.ds(col * col_len, col_len)], tokv)

    def _count(i, acc):
        tv = tokv[pl.ds(i * _LANES, _LANES)]
        return acc + plsc.all_reduce_population_count(tv == _SEP_TOKEN_IDX)

    base = lax.fori_loop(
        0, r * nvec, _count, jnp.zeros((_LANES,), jnp.int32)
    )

    def _body(i, carry):
        tv = tokv[pl.ds((r * nvec + i) * _LANES, _LANES)]
        m = tv == _SEP_TOKEN_IDX
        si = m.astype(jnp.int32)
        seg = plsc.cumsum(si) - si + carry
        segv[pl.ds(i * _LANES, _LANES)] = jnp.minimum(seg, 2).astype(jnp.float32)
        return carry + plsc.all_reduce_population_count(m)

    lax.fori_loop(0, nvec, _body, base)
    pltpu.sync_copy(segv, out_ref.at[pl.ds(w * _CHUNK, _CHUNK)])


def _tc_add_kernel(seg_ref, x_ref, emb_ref, out_ref):
    seg = jnp.transpose(seg_ref[...])
    nseg, hid = emb_ref.shape
    seg3 = seg[:, :, None]
    e0 = emb_ref[0, :].reshape(1, 1, hid)
    e1 = emb_ref[1, :].reshape(1, 1, hid)
    e2 = emb_ref[2, :].reshape(1, 1, hid)
    emb = jnp.where(seg3 == 0, e0, jnp.where(seg3 == 1, e1, e2))
    out_ref[...] = x_ref[...] + emb


def kernel(x, tokens, emb_table):
    seq, batch, hid = x.shape
    nseg = emb_table.shape[0]
    n = seq * batch
    tok_t = jnp.transpose(tokens.astype(jnp.int32)).reshape(n)

    sc = pl.kernel(
        _sc_seg_ids,
        mesh=plsc.VectorSubcoreMesh(core_axis_name="c", subcore_axis_name="s"),
        compiler_params=pltpu.CompilerParams(needs_layout_passes=False),
        out_type=jax.ShapeDtypeStruct((n,), jnp.float32),
        scratch_types=[
            pltpu.VMEM((_CHUNK * _CHUNKS_PER_COL,), jnp.int32),
            pltpu.VMEM((_CHUNK,), jnp.float32),
        ],
    )
    seg_t = sc(tok_t).reshape(batch, seq)

    grid = seq // _SEQ_BLK
    return pl.pallas_call(
        _tc_add_kernel,
        grid=(grid,),
        compiler_params=pltpu.CompilerParams(
            dimension_semantics=("parallel",)
        ),
        in_specs=[
            pl.BlockSpec((batch, _SEQ_BLK), lambda i: (0, i)),
            pl.BlockSpec((_SEQ_BLK, batch, hid), lambda i: (i, 0, 0)),
            pl.BlockSpec((nseg, hid), lambda i: (0, 0)),
        ],
        out_specs=pl.BlockSpec((_SEQ_BLK, batch, hid), lambda i: (i, 0, 0)),
        out_shape=jax.ShapeDtypeStruct(x.shape, x.dtype),
    )(seg_t, x, emb_table)


# speedup vs baseline: 1.0542x; 1.0165x over previous
import functools

import jax
import jax.numpy as jnp
from jax import lax
from jax.experimental import pallas as pl
from jax.experimental.pallas import tpu as pltpu
from jax.experimental.pallas import tpu_sc as plsc

_SEP_TOKEN_IDX = 5
_SEQ_BLK = 512
_LANES = 16
_NW = 32
_CHUNK = 1024
_CHUNKS_PER_COL = 8


def _sc_seg_ids(tok_ref, out_ref, tokv, segv):
    c = lax.axis_index("c")
    s = lax.axis_index("s")
    w = c * 16 + s
    nvec = _CHUNK // _LANES
    col = w // _CHUNKS_PER_COL
    r = w % _CHUNKS_PER_COL
    col_len = _CHUNK * _CHUNKS_PER_COL

    pltpu.sync_copy(tok_ref.at[pl.ds(col * col_len, col_len)], tokv)

    def _count(i, acc):
        b = i * 4 * _LANES
        for k in range(4):
            tv = tokv[pl.ds(b + k * _LANES, _LANES)]
            acc = acc + plsc.all_reduce_population_count(tv == _SEP_TOKEN_IDX)
        return acc

    base = lax.fori_loop(
        0, r * (nvec // 4), _count, jnp.zeros((_LANES,), jnp.int32)
    )

    def _body(i, carry):
        tv = tokv[pl.ds((r * nvec + i) * _LANES, _LANES)]
        m = tv == _SEP_TOKEN_IDX
        si = m.astype(jnp.int32)
        seg = plsc.cumsum(si) - si + carry
        segv[pl.ds(i * _LANES, _LANES)] = jnp.minimum(seg, 2).astype(jnp.float32)
        return carry + plsc.all_reduce_population_count(m)

    lax.fori_loop(0, nvec, _body, base)
    pltpu.sync_copy(segv, out_ref.at[pl.ds(w * _CHUNK, _CHUNK)])


def _tc_add_kernel(seg_ref, x_ref, emb_ref, out_ref):
    seg = jnp.transpose(seg_ref[...])
    nseg, hid = emb_ref.shape
    seg3 = seg[:, :, None]
    e0 = emb_ref[0, :].reshape(1, 1, hid)
    e1 = emb_ref[1, :].reshape(1, 1, hid)
    e2 = emb_ref[2, :].reshape(1, 1, hid)
    emb = jnp.where(seg3 == 0, e0, jnp.where(seg3 == 1, e1, e2))
    out_ref[...] = x_ref[...] + emb


def kernel(x, tokens, emb_table):
    seq, batch, hid = x.shape
    nseg = emb_table.shape[0]
    n = seq * batch
    tok_t = jnp.transpose(tokens.astype(jnp.int32)).reshape(n)

    sc = pl.kernel(
        _sc_seg_ids,
        mesh=plsc.VectorSubcoreMesh(core_axis_name="c", subcore_axis_name="s"),
        compiler_params=pltpu.CompilerParams(needs_layout_passes=False),
        out_type=jax.ShapeDtypeStruct((n,), jnp.float32),
        scratch_types=[
            pltpu.VMEM((_CHUNK * _CHUNKS_PER_COL,), jnp.int32),
            pltpu.VMEM((_CHUNK,), jnp.float32),
        ],
    )
    seg_t = sc(tok_t).reshape(batch, seq)

    grid = seq // _SEQ_BLK
    return pl.pallas_call(
        _tc_add_kernel,
        grid=(grid,),
        compiler_params=pltpu.CompilerParams(
            dimension_semantics=("parallel",)
        ),
        in_specs=[
            pl.BlockSpec((batch, _SEQ_BLK), lambda i: (0, i)),
            pl.BlockSpec((_SEQ_BLK, batch, hid), lambda i: (i, 0, 0)),
            pl.BlockSpec((nseg, hid), lambda i: (0, 0)),
        ],
        out_specs=pl.BlockSpec((_SEQ_BLK, batch, hid), lambda i: (i, 0, 0)),
        out_shape=jax.ShapeDtypeStruct(x.shape, x.dtype),
    )(seg_t, x, emb_table)
